# baseline (speedup 1.0000x reference)
import functools

import jax
import jax.numpy as jnp
from jax import lax
from jax.experimental import pallas as pl
from jax.experimental.pallas import tpu as pltpu
from jax.experimental.pallas import tpu_sc as plsc

BATCH = 16384
EMB = 32
NC, NS, L = 2, 16, 16
NW = NC * NS
BPW = BATCH // NW
NG = BPW // L

_mesh = plsc.VectorSubcoreMesh(core_axis_name="c", subcore_axis_name="s")


@functools.partial(
    pl.kernel,
    mesh=_mesh,
    compiler_params=pltpu.CompilerParams(use_tc_tiling_on_sc=False),
    out_type=jax.ShapeDtypeStruct((BATCH,), jnp.float32),
    scratch_types=[
        pltpu.VMEM((BPW,), jnp.int32),
        pltpu.VMEM((BPW,), jnp.int32),
        pltpu.VMEM((BPW, EMB), jnp.float32),
        pltpu.VMEM((BPW, EMB), jnp.float32),
        pltpu.VMEM((BPW,), jnp.float32),
        pltpu.SemaphoreType.DMA,
    ],
)
def _sc_scores(users_hbm, items_hbm, uemb_hbm, iemb_hbm, out_hbm,
               uidx_v, iidx_v, urows_v, irows_v, score_v, sem):
    wid = lax.axis_index("s") * NC + lax.axis_index("c")
    base = wid * BPW
    pltpu.sync_copy(users_hbm.at[pl.ds(base, BPW)], uidx_v)
    pltpu.sync_copy(items_hbm.at[pl.ds(base, BPW)], iidx_v)
    cu = pltpu.async_copy(uemb_hbm.at[uidx_v], urows_v, sem)
    ci = pltpu.async_copy(iemb_hbm.at[iidx_v], irows_v, sem)
    cu.wait()
    ci.wait()

    lane = lax.iota(jnp.int32, L)
    bitrev = [int(f"{j:04b}"[::-1], 2) for j in range(L)]

    _dnums = lax.GatherDimensionNumbers(
        offset_dims=(), collapsed_slice_dims=(0,), start_index_map=(0,))

    def permute(x, idx):
        return lax.gather(x, idx[:, None], dimension_numbers=_dnums,
                          slice_sizes=(1,),
                          mode=lax.GatherScatterMode.PROMISE_IN_BOUNDS)

    def group(g, _):
        vs = []
        for j in range(L):
            r = g * L + bitrev[j]
            u0 = urows_v[r, pl.ds(0, L)]
            u1 = urows_v[r, pl.ds(L, L)]
            i0 = irows_v[r, pl.ds(0, L)]
            i1 = irows_v[r, pl.ds(L, L)]
            vs.append(u0 * i0 + u1 * i1)
        for d in (8, 4, 2, 1):
            perm = lane ^ d
            keep = (lane & d) == 0
            nxt = []
            for k in range(len(vs) // 2):
                a, b = vs[2 * k], vs[2 * k + 1]
                fa = a + permute(a, perm)
                fb = b + permute(b, perm)
                nxt.append(jnp.where(keep, fa, fb))
            vs = nxt
        score_v[pl.ds(g * L, L)] = vs[0]
        return 0

    lax.fori_loop(0, NG, group, 0)

    pltpu.sync_copy(score_v, out_hbm.at[pl.ds(base, BPW)])


def kernel(users, items, user_emb, item_emb):
    return _sc_scores(users.astype(jnp.int32), items.astype(jnp.int32),
                      user_emb, item_emb)
